# baseline (speedup 1.0000x reference)
import functools

import jax
import jax.numpy as jnp
from jax import lax
from jax.experimental import pallas as pl
from jax.experimental.pallas import tpu as pltpu
from jax.experimental.pallas import tpu_sc as plsc

N = 10000
E = 320000
IN_DIM = 128
HID = 128
OUT_DIM = 64
W64 = 64

C = 80
NCHUNK = E // C
NSUB = 16
CPT = NCHUNK // NSUB
ZPT = 632
NPAD = NSUB * ZPT

_MESH = plsc.VectorSubcoreMesh(core_axis_name="c", subcore_axis_name="s")
_SC_PARAMS = pltpu.CompilerParams(
    needs_layout_passes=False, use_tc_tiling_on_sc=False
)


@functools.partial(
    pl.kernel,
    out_type=jax.ShapeDtypeStruct((2, NSUB, NPAD), jnp.float32),
    mesh=_MESH,
    compiler_params=_SC_PARAMS,
    scratch_types=[
        pltpu.VMEM((CPT, C), jnp.int32),
        pltpu.VMEM((NPAD,), jnp.float32),
    ],
)
def _sc_degree(dst2_hbm, deg_hbm, idxd, hist):
    c = lax.axis_index("c")
    s = lax.axis_index("s")
    pltpu.sync_copy(dst2_hbm.at[c, s], idxd)

    def zero(r, _):
        hist[pl.ds(r * 16, 16)] = jnp.zeros((16,), jnp.float32)
        return ()
    lax.fori_loop(0, NPAD // 16, zero, ())

    ones16 = jnp.ones((16,), jnp.float32)

    def body(j, _):
        for k in range(C // 16):
            idx = idxd[j, pl.ds(k * 16, 16)]
            plsc.addupdate_scatter(hist, [idx], ones16)
        return ()
    lax.fori_loop(0, CPT, body, ())
    pltpu.sync_copy(hist, deg_hbm.at[c, s])


@functools.partial(
    pl.kernel,
    out_type=jax.ShapeDtypeStruct((2, NPAD, W64), jnp.float32),
    mesh=_MESH,
    compiler_params=_SC_PARAMS,
    scratch_types=[
        pltpu.VMEM((CPT + 5, C), jnp.int32),
        pltpu.VMEM((CPT, C), jnp.int32),
        pltpu.VMEM((5, C, W64), jnp.float32),
        pltpu.VMEM_SHARED((NPAD, W64), jnp.float32),
        pltpu.SemaphoreType.DMA,
        pltpu.SemaphoreType.DMA,
        pltpu.SemaphoreType.DMA,
        pltpu.SemaphoreType.DMA,
        pltpu.SemaphoreType.DMA,
    ],
)
def _sc_agg(src4_hbm, dst4_hbm, table_hbm, agg_hbm, idxs, idxd, rows, acc,
            g0, g1, g2, g3, g4):
    c = lax.axis_index("c")
    s = lax.axis_index("s")
    sems = (g0, g1, g2, g3, g4)
    pltpu.sync_copy(src4_hbm.at[c, s], idxs.at[pl.ds(0, CPT)])
    pltpu.sync_copy(dst4_hbm.at[c, s], idxd)

    def prow(r, _):
        for k in range(C // 16):
            idxs[CPT + r, pl.ds(k * 16, 16)] = jnp.zeros((16,), jnp.int32)
        return ()
    lax.fori_loop(0, 5, prow, ())

    def zrow(r, _):
        for k in range(W64 // 16):
            rows[0, r, pl.ds(k * 16, 16)] = jnp.zeros((16,), jnp.float32)
        return ()
    lax.fori_loop(0, C, zrow, ())
    nfull = ZPT // C
    rem = ZPT - nfull * C
    for k in range(nfull):
        pltpu.sync_copy(rows.at[0], acc.at[pl.ds(s * ZPT + k * C, C)])
    pltpu.sync_copy(rows.at[0].at[pl.ds(0, rem)],
                    acc.at[pl.ds(s * ZPT + nfull * C, rem)])
    plsc.subcore_barrier()

    for b in range(5):
        pltpu.async_copy(table_hbm.at[idxs.at[b]], rows.at[b], sems[b])

    def body(t, _):
        j = t * 5
        for b in range(5):
            pltpu.make_async_copy(
                table_hbm.at[idxs.at[j + b]], rows.at[b], sems[b]
            ).wait()
            pltpu.sync_copy(rows.at[b], acc.at[idxd.at[j + b]], add=True)
            pltpu.async_copy(
                table_hbm.at[idxs.at[j + b + 5]], rows.at[b], sems[b]
            )
        return ()
    lax.fori_loop(0, CPT // 5, body, ())
    for b in range(5):
        pltpu.make_async_copy(
            table_hbm.at[idxs.at[CPT + b]], rows.at[b], sems[b]
        ).wait()
    plsc.subcore_barrier()
    pltpu.sync_copy(acc.at[pl.ds(s * ZPT, ZPT)],
                    agg_hbm.at[c, pl.ds(s * ZPT, ZPT)])


def _tc1_body(deg_ref, x_ref, w1a_ref, w1b_ref, hs_ref, dinv_ref):
    deg = jnp.sum(deg_ref[...], axis=2)
    dinv = lax.rsqrt(deg + 1.0)[:, :, None]
    dinv_ref[...] = dinv
    x = x_ref[...]
    ha = dinv[0] * jnp.dot(x, w1a_ref[...], preferred_element_type=jnp.float32)
    hb = dinv[1] * jnp.dot(x, w1b_ref[...], preferred_element_type=jnp.float32)
    hs_ref[...] = jnp.stack([
        jnp.stack([ha[:, :W64], ha[:, W64:]]),
        jnp.stack([hb[:, :W64], hb[:, W64:]]),
    ])


def _tc2_body(agg1a_ref, agg1b_ref, hs1_ref, dinv_ref, b1_ref,
              w2a_ref, w2b_ref, hs2_ref):
    dinv = dinv_ref[...]
    agg_a = jnp.concatenate([agg1a_ref[0], agg1a_ref[1]], axis=-1)
    agg_b = jnp.concatenate([agg1b_ref[0], agg1b_ref[1]], axis=-1)
    hs_a = jnp.concatenate([hs1_ref[0, 0], hs1_ref[0, 1]], axis=-1)
    hs_b = jnp.concatenate([hs1_ref[1, 0], hs1_ref[1, 1]], axis=-1)
    xa = jax.nn.relu(dinv[0] * (agg_a + hs_a) + b1_ref[0])
    xb = jax.nn.relu(dinv[1] * (agg_b + hs_b) + b1_ref[1])
    ha = jnp.dot(xa, w2a_ref[...], preferred_element_type=jnp.float32)
    hb = jnp.dot(xb, w2b_ref[...], preferred_element_type=jnp.float32)
    hs2_ref[...] = jnp.stack([dinv[0] * ha, dinv[1] * hb])


def _tc3_body(agg_ref, hs_ref, dinv_ref, b2_ref, logits_ref, la_ref, lb_ref):
    l = dinv_ref[...] * (agg_ref[...] + hs_ref[...]) + b2_ref[...]
    la_ref[...] = l[0]
    lb_ref[...] = l[1]
    logits_ref[...] = 0.5 * l[0] + 0.5 * l[1]


_BLK = 1000
_GRID = N // _BLK


def _full2(shape):
    return pl.BlockSpec(shape, lambda i: (0, 0))


def _rows3(width):
    return pl.BlockSpec((2, _BLK, width), lambda i: (0, i, 0))


def _rows2(width):
    return pl.BlockSpec((_BLK, width), lambda i: (i, 0))


def _edges4(e0, e1):
    return jnp.stack([e0, e1]).reshape(2, NSUB, CPT, C)


def kernel(x, edge_a, edge_b, W1a, b1a, W2a, b2a, W1b, b1b, W2b, b2b):
    sa, da = edge_a[0], edge_a[1]
    sb, db = edge_b[0], edge_b[1]
    srcL1a = _edges4(sa, sa + N)
    dstL1a = _edges4(da, da)
    srcL1b = _edges4(sb, sb + N)
    dstL1b = _edges4(db, db)
    srcL2 = _edges4(sa, sb + N)
    dstL2 = _edges4(da, db)
    dst2 = _edges4(da, db)

    degp = _sc_degree(dst2)[:, :, :N].transpose(0, 2, 1)

    hs1, dinv = pl.pallas_call(
        _tc1_body,
        grid=(_GRID,),
        in_specs=[pl.BlockSpec((2, _BLK, NSUB), lambda i: (0, i, 0)),
                  _rows2(IN_DIM), _full2((IN_DIM, HID)),
                  _full2((IN_DIM, HID))],
        out_specs=[pl.BlockSpec((2, 2, _BLK, W64), lambda i: (0, 0, i, 0)),
                   _rows3(1)],
        out_shape=[jax.ShapeDtypeStruct((2, 2, N, W64), jnp.float32),
                   jax.ShapeDtypeStruct((2, N, 1), jnp.float32)],
    )(degp, x, W1a, W1b)

    agg1a = _sc_agg(srcL1a, dstL1a, hs1[0].reshape(2 * N, W64))[:, :N]
    agg1b = _sc_agg(srcL1b, dstL1b, hs1[1].reshape(2 * N, W64))[:, :N]

    b1 = jnp.stack([b1a, b1b]).reshape(2, 1, HID)
    hs2 = pl.pallas_call(
        _tc2_body,
        grid=(_GRID,),
        in_specs=[_rows3(W64), _rows3(W64),
                  pl.BlockSpec((2, 2, _BLK, W64), lambda i: (0, 0, i, 0)),
                  _rows3(1),
                  pl.BlockSpec((2, 1, HID), lambda i: (0, 0, 0)),
                  _full2((HID, OUT_DIM)), _full2((HID, OUT_DIM))],
        out_specs=_rows3(OUT_DIM),
        out_shape=jax.ShapeDtypeStruct((2, N, OUT_DIM), jnp.float32),
    )(agg1a, agg1b, hs1, dinv, b1, W2a, W2b)

    agg2 = _sc_agg(srcL2, dstL2, hs2.reshape(2 * N, OUT_DIM))[:, :N]

    b2 = jnp.stack([b2a, b2b]).reshape(2, 1, OUT_DIM)
    logits, la, lb = pl.pallas_call(
        _tc3_body,
        grid=(_GRID,),
        in_specs=[_rows3(OUT_DIM), _rows3(OUT_DIM), _rows3(1),
                  pl.BlockSpec((2, 1, OUT_DIM), lambda i: (0, 0, 0))],
        out_specs=[_rows2(OUT_DIM)] * 3,
        out_shape=[jax.ShapeDtypeStruct((N, OUT_DIM), jnp.float32)] * 3,
    )(agg2, hs2, dinv, b2)

    return (logits, la, lb)


# speedup vs baseline: 1.3754x; 1.3754x over previous
import functools

import jax
import jax.numpy as jnp
from jax import lax
from jax.experimental import pallas as pl
from jax.experimental.pallas import tpu as pltpu
from jax.experimental.pallas import tpu_sc as plsc

N = 10000
E = 320000
IN_DIM = 128
HID = 128
OUT_DIM = 64
W64 = 64

C = 125
NCHUNK = E // C
NSUB = 16
CPT = NCHUNK // NSUB
CD = 80
CPTD = (E // CD) // NSUB
ZPT = 632
NPAD = NSUB * ZPT

_MESH = plsc.VectorSubcoreMesh(core_axis_name="c", subcore_axis_name="s")
_SC_PARAMS = pltpu.CompilerParams(
    needs_layout_passes=False, use_tc_tiling_on_sc=False
)


@functools.partial(
    pl.kernel,
    out_type=jax.ShapeDtypeStruct((2, NSUB, NPAD), jnp.float32),
    mesh=_MESH,
    compiler_params=_SC_PARAMS,
    scratch_types=[
        pltpu.VMEM((CPTD, CD), jnp.int32),
        pltpu.VMEM((NPAD,), jnp.float32),
    ],
)
def _sc_degree(dst2_hbm, deg_hbm, idxd, hist):
    c = lax.axis_index("c")
    s = lax.axis_index("s")
    pltpu.sync_copy(dst2_hbm.at[c, s], idxd)

    def zero(r, _):
        hist[pl.ds(r * 16, 16)] = jnp.zeros((16,), jnp.float32)
        return ()
    lax.fori_loop(0, NPAD // 16, zero, ())

    ones16 = jnp.ones((16,), jnp.float32)

    def body(j, _):
        for k in range(CD // 16):
            idx = idxd[j, pl.ds(k * 16, 16)]
            plsc.addupdate_scatter(hist, [idx], ones16)
        return ()
    lax.fori_loop(0, CPTD, body, ())
    pltpu.sync_copy(hist, deg_hbm.at[c, s])


@functools.partial(
    pl.kernel,
    out_type=jax.ShapeDtypeStruct((2, NPAD, W64), jnp.float32),
    mesh=_MESH,
    compiler_params=_SC_PARAMS,
    scratch_types=[
        pltpu.VMEM((CPT, C), jnp.int32),
        pltpu.VMEM((CPT, C), jnp.int32),
        pltpu.VMEM((C, W64), jnp.float32),
        pltpu.VMEM_SHARED((NPAD, W64), jnp.float32),
        pltpu.SemaphoreType.DMA,
    ],
)
def _sc_agg(src4_hbm, dst4_hbm, table_hbm, agg_hbm, idxs, idxd, rows, acc, sem):
    c = lax.axis_index("c")
    s = lax.axis_index("s")
    pltpu.sync_copy(src4_hbm.at[c, s], idxs)
    pltpu.sync_copy(dst4_hbm.at[c, s], idxd)

    def zrow(r, _):
        for k in range(W64 // 16):
            rows[r, pl.ds(k * 16, 16)] = jnp.zeros((16,), jnp.float32)
        return ()
    lax.fori_loop(0, C, zrow, ())
    nfull = ZPT // C
    rem = ZPT - nfull * C
    for k in range(nfull):
        pltpu.sync_copy(rows, acc.at[pl.ds(s * ZPT + k * C, C)])
    pltpu.sync_copy(rows.at[pl.ds(0, rem)],
                    acc.at[pl.ds(s * ZPT + nfull * C, rem)])
    plsc.subcore_barrier()

    def body(j, _):
        pltpu.async_copy(table_hbm.at[idxs.at[j]], rows, sem).wait()
        pltpu.sync_copy(rows, acc.at[idxd.at[j]], add=True)
        return ()
    lax.fori_loop(0, CPT, body, ())
    plsc.subcore_barrier()
    pltpu.sync_copy(acc.at[pl.ds(s * ZPT, ZPT)],
                    agg_hbm.at[c, pl.ds(s * ZPT, ZPT)])


def _tc1_body(deg_ref, x_ref, w1a_ref, w1b_ref, hs_ref, dinv_ref):
    deg = jnp.sum(deg_ref[...], axis=2)
    dinv = lax.rsqrt(deg + 1.0)[:, :, None]
    dinv_ref[...] = dinv
    x = x_ref[...]
    ha = dinv[0] * jnp.dot(x, w1a_ref[...], preferred_element_type=jnp.float32)
    hb = dinv[1] * jnp.dot(x, w1b_ref[...], preferred_element_type=jnp.float32)
    hs_ref[...] = jnp.stack([
        jnp.stack([ha[:, :W64], ha[:, W64:]]),
        jnp.stack([hb[:, :W64], hb[:, W64:]]),
    ])


def _tc2_body(agg1a_ref, agg1b_ref, hs1_ref, dinv_ref, b1_ref,
              w2a_ref, w2b_ref, hs2_ref):
    dinv = dinv_ref[...]
    agg_a = jnp.concatenate([agg1a_ref[0], agg1a_ref[1]], axis=-1)
    agg_b = jnp.concatenate([agg1b_ref[0], agg1b_ref[1]], axis=-1)
    hs_a = jnp.concatenate([hs1_ref[0, 0], hs1_ref[0, 1]], axis=-1)
    hs_b = jnp.concatenate([hs1_ref[1, 0], hs1_ref[1, 1]], axis=-1)
    xa = jax.nn.relu(dinv[0] * (agg_a + hs_a) + b1_ref[0])
    xb = jax.nn.relu(dinv[1] * (agg_b + hs_b) + b1_ref[1])
    ha = jnp.dot(xa, w2a_ref[...], preferred_element_type=jnp.float32)
    hb = jnp.dot(xb, w2b_ref[...], preferred_element_type=jnp.float32)
    hs2_ref[...] = jnp.stack([dinv[0] * ha, dinv[1] * hb])


def _tc3_body(agg_ref, hs_ref, dinv_ref, b2_ref, logits_ref, la_ref, lb_ref):
    l = dinv_ref[...] * (agg_ref[...] + hs_ref[...]) + b2_ref[...]
    la_ref[...] = l[0]
    lb_ref[...] = l[1]
    logits_ref[...] = 0.5 * l[0] + 0.5 * l[1]


_BLK = 1000
_GRID = N // _BLK


def _full2(shape):
    return pl.BlockSpec(shape, lambda i: (0, 0))


def _rows3(width):
    return pl.BlockSpec((2, _BLK, width), lambda i: (0, i, 0))


def _rows2(width):
    return pl.BlockSpec((_BLK, width), lambda i: (i, 0))


def _edges4(e0, e1):
    return jnp.stack([e0, e1]).reshape(2, NSUB, CPT, C)


def _edges4d(e0, e1):
    return jnp.stack([e0, e1]).reshape(2, NSUB, CPTD, CD)


def kernel(x, edge_a, edge_b, W1a, b1a, W2a, b2a, W1b, b1b, W2b, b2b):
    sa, da = edge_a[0], edge_a[1]
    sb, db = edge_b[0], edge_b[1]
    srcL1a = _edges4(sa, sa + N)
    dstL1a = _edges4(da, da)
    srcL1b = _edges4(sb, sb + N)
    dstL1b = _edges4(db, db)
    srcL2 = _edges4(sa, sb + N)
    dstL2 = _edges4(da, db)
    dst2 = _edges4d(da, db)

    degp = _sc_degree(dst2)[:, :, :N].transpose(0, 2, 1)

    hs1, dinv = pl.pallas_call(
        _tc1_body,
        grid=(_GRID,),
        in_specs=[pl.BlockSpec((2, _BLK, NSUB), lambda i: (0, i, 0)),
                  _rows2(IN_DIM), _full2((IN_DIM, HID)),
                  _full2((IN_DIM, HID))],
        out_specs=[pl.BlockSpec((2, 2, _BLK, W64), lambda i: (0, 0, i, 0)),
                   _rows3(1)],
        out_shape=[jax.ShapeDtypeStruct((2, 2, N, W64), jnp.float32),
                   jax.ShapeDtypeStruct((2, N, 1), jnp.float32)],
    )(degp, x, W1a, W1b)

    agg1a = _sc_agg(srcL1a, dstL1a, hs1[0].reshape(2 * N, W64))[:, :N]
    agg1b = _sc_agg(srcL1b, dstL1b, hs1[1].reshape(2 * N, W64))[:, :N]

    b1 = jnp.stack([b1a, b1b]).reshape(2, 1, HID)
    hs2 = pl.pallas_call(
        _tc2_body,
        grid=(_GRID,),
        in_specs=[_rows3(W64), _rows3(W64),
                  pl.BlockSpec((2, 2, _BLK, W64), lambda i: (0, 0, i, 0)),
                  _rows3(1),
                  pl.BlockSpec((2, 1, HID), lambda i: (0, 0, 0)),
                  _full2((HID, OUT_DIM)), _full2((HID, OUT_DIM))],
        out_specs=_rows3(OUT_DIM),
        out_shape=jax.ShapeDtypeStruct((2, N, OUT_DIM), jnp.float32),
    )(agg1a, agg1b, hs1, dinv, b1, W2a, W2b)

    agg2 = _sc_agg(srcL2, dstL2, hs2.reshape(2 * N, OUT_DIM))[:, :N]

    b2 = jnp.stack([b2a, b2b]).reshape(2, 1, OUT_DIM)
    logits, la, lb = pl.pallas_call(
        _tc3_body,
        grid=(_GRID,),
        in_specs=[_rows3(OUT_DIM), _rows3(OUT_DIM), _rows3(1),
                  pl.BlockSpec((2, 1, OUT_DIM), lambda i: (0, 0, 0))],
        out_specs=[_rows2(OUT_DIM)] * 3,
        out_shape=[jax.ShapeDtypeStruct((N, OUT_DIM), jnp.float32)] * 3,
    )(agg2, hs2, dinv, b2)

    return (logits, la, lb)


# speedup vs baseline: 1.8420x; 1.3392x over previous
import functools

import jax
import jax.numpy as jnp
from jax import lax
from jax.experimental import pallas as pl
from jax.experimental.pallas import tpu as pltpu
from jax.experimental.pallas import tpu_sc as plsc

N = 10000
E = 320000
IN_DIM = 128
HID = 128
OUT_DIM = 64
W64 = 64

C = 125
NCHUNK = E // C
NSUB = 16
CPT = NCHUNK // NSUB
CD = 80
CPTD = (E // CD) // NSUB
GC = 400
NSEG = 20000
ZPT = 632
NPAD = NSUB * ZPT

_MESH = plsc.VectorSubcoreMesh(core_axis_name="c", subcore_axis_name="s")
_SC_PARAMS = pltpu.CompilerParams(
    needs_layout_passes=False, use_tc_tiling_on_sc=False
)


@functools.partial(
    pl.kernel,
    out_type=jax.ShapeDtypeStruct((2, NSUB, NPAD), jnp.float32),
    mesh=_MESH,
    compiler_params=_SC_PARAMS,
    scratch_types=[
        pltpu.VMEM((CPTD, CD), jnp.int32),
        pltpu.VMEM((NPAD,), jnp.float32),
    ],
)
def _sc_degree(dst2_hbm, deg_hbm, idxd, hist):
    c = lax.axis_index("c")
    s = lax.axis_index("s")
    pltpu.sync_copy(dst2_hbm.at[c, s], idxd)

    def zero(r, _):
        hist[pl.ds(r * 16, 16)] = jnp.zeros((16,), jnp.float32)
        return ()
    lax.fori_loop(0, NPAD // 16, zero, ())

    ones16 = jnp.ones((16,), jnp.float32)

    def body(j, _):
        for k in range(CD // 16):
            idx = idxd[j, pl.ds(k * 16, 16)]
            plsc.addupdate_scatter(hist, [idx], ones16)
        return ()
    lax.fori_loop(0, CPTD, body, ())
    pltpu.sync_copy(hist, deg_hbm.at[c, s])


@functools.partial(
    pl.kernel,
    out_type=jax.ShapeDtypeStruct((2, NPAD, W64), jnp.float32),
    mesh=_MESH,
    compiler_params=_SC_PARAMS,
    scratch_types=[
        pltpu.VMEM((NSEG,), jnp.int32),
        pltpu.VMEM((NSEG,), jnp.int32),
        pltpu.VMEM((GC, W64), jnp.float32),
        pltpu.VMEM_SHARED((NPAD, W64), jnp.float32),
        pltpu.SemaphoreType.DMA,
    ],
)
def _sc_agg(src4_hbm, dst4_hbm, table_hbm, agg_hbm, idxs, idxd, rows, acc, sem):
    c = lax.axis_index("c")
    s = lax.axis_index("s")
    pltpu.sync_copy(src4_hbm.at[c, s], idxs)
    pltpu.sync_copy(dst4_hbm.at[c, s], idxd)

    def zrow(r, _):
        for k in range(W64 // 16):
            rows[r, pl.ds(k * 16, 16)] = jnp.zeros((16,), jnp.float32)
        return ()
    lax.fori_loop(0, GC, zrow, ())
    nfull = ZPT // GC
    rem = ZPT - nfull * GC
    for k in range(nfull):
        pltpu.sync_copy(rows, acc.at[pl.ds(s * ZPT + k * GC, GC)])
    pltpu.sync_copy(rows.at[pl.ds(0, rem)],
                    acc.at[pl.ds(s * ZPT + nfull * GC, rem)])
    plsc.subcore_barrier()

    def body(t, _):
        j = t * GC
        pltpu.async_copy(
            table_hbm.at[idxs.at[pl.ds(j, GC)]], rows, sem
        ).wait()
        pltpu.sync_copy(rows, acc.at[idxd.at[pl.ds(j, GC)]], add=True)
        return ()
    lax.fori_loop(0, NSEG // GC, body, ())
    plsc.subcore_barrier()
    pltpu.sync_copy(acc.at[pl.ds(s * ZPT, ZPT)],
                    agg_hbm.at[c, pl.ds(s * ZPT, ZPT)])


def _tc1_body(deg_ref, x_ref, w1a_ref, w1b_ref, hs_ref, dinv_ref):
    deg = jnp.sum(deg_ref[...], axis=2)
    dinv = lax.rsqrt(deg + 1.0)[:, :, None]
    dinv_ref[...] = dinv
    x = x_ref[...]
    ha = dinv[0] * jnp.dot(x, w1a_ref[...], preferred_element_type=jnp.float32)
    hb = dinv[1] * jnp.dot(x, w1b_ref[...], preferred_element_type=jnp.float32)
    hs_ref[...] = jnp.stack([
        jnp.stack([ha[:, :W64], ha[:, W64:]]),
        jnp.stack([hb[:, :W64], hb[:, W64:]]),
    ])


def _tc2_body(agg1a_ref, agg1b_ref, hs1_ref, dinv_ref, b1_ref,
              w2a_ref, w2b_ref, hs2_ref):
    dinv = dinv_ref[...]
    agg_a = jnp.concatenate([agg1a_ref[0], agg1a_ref[1]], axis=-1)
    agg_b = jnp.concatenate([agg1b_ref[0], agg1b_ref[1]], axis=-1)
    hs_a = jnp.concatenate([hs1_ref[0, 0], hs1_ref[0, 1]], axis=-1)
    hs_b = jnp.concatenate([hs1_ref[1, 0], hs1_ref[1, 1]], axis=-1)
    xa = jax.nn.relu(dinv[0] * (agg_a + hs_a) + b1_ref[0])
    xb = jax.nn.relu(dinv[1] * (agg_b + hs_b) + b1_ref[1])
    ha = jnp.dot(xa, w2a_ref[...], preferred_element_type=jnp.float32)
    hb = jnp.dot(xb, w2b_ref[...], preferred_element_type=jnp.float32)
    hs2_ref[...] = jnp.stack([dinv[0] * ha, dinv[1] * hb])


def _tc3_body(agg_ref, hs_ref, dinv_ref, b2_ref, logits_ref, la_ref, lb_ref):
    l = dinv_ref[...] * (agg_ref[...] + hs_ref[...]) + b2_ref[...]
    la_ref[...] = l[0]
    lb_ref[...] = l[1]
    logits_ref[...] = 0.5 * l[0] + 0.5 * l[1]


_BLK = 1000
_GRID = N // _BLK


def _full2(shape):
    return pl.BlockSpec(shape, lambda i: (0, 0))


def _rows3(width):
    return pl.BlockSpec((2, _BLK, width), lambda i: (0, i, 0))


def _rows2(width):
    return pl.BlockSpec((_BLK, width), lambda i: (i, 0))


def _edges4(e0, e1):
    return jnp.stack([e0, e1]).reshape(2, NSUB, NSEG)


def _edges4d(e0, e1):
    return jnp.stack([e0, e1]).reshape(2, NSUB, CPTD, CD)


def kernel(x, edge_a, edge_b, W1a, b1a, W2a, b2a, W1b, b1b, W2b, b2b):
    sa, da = edge_a[0], edge_a[1]
    sb, db = edge_b[0], edge_b[1]
    srcL1a = _edges4(sa, sa + N)
    dstL1a = _edges4(da, da)
    srcL1b = _edges4(sb, sb + N)
    dstL1b = _edges4(db, db)
    srcL2 = _edges4(sa, sb + N)
    dstL2 = _edges4(da, db)
    dst2 = _edges4d(da, db)

    degp = _sc_degree(dst2)[:, :, :N].transpose(0, 2, 1)

    hs1, dinv = pl.pallas_call(
        _tc1_body,
        grid=(_GRID,),
        in_specs=[pl.BlockSpec((2, _BLK, NSUB), lambda i: (0, i, 0)),
                  _rows2(IN_DIM), _full2((IN_DIM, HID)),
                  _full2((IN_DIM, HID))],
        out_specs=[pl.BlockSpec((2, 2, _BLK, W64), lambda i: (0, 0, i, 0)),
                   _rows3(1)],
        out_shape=[jax.ShapeDtypeStruct((2, 2, N, W64), jnp.float32),
                   jax.ShapeDtypeStruct((2, N, 1), jnp.float32)],
    )(degp, x, W1a, W1b)

    agg1a = _sc_agg(srcL1a, dstL1a, hs1[0].reshape(2 * N, W64))[:, :N]
    agg1b = _sc_agg(srcL1b, dstL1b, hs1[1].reshape(2 * N, W64))[:, :N]

    b1 = jnp.stack([b1a, b1b]).reshape(2, 1, HID)
    hs2 = pl.pallas_call(
        _tc2_body,
        grid=(_GRID,),
        in_specs=[_rows3(W64), _rows3(W64),
                  pl.BlockSpec((2, 2, _BLK, W64), lambda i: (0, 0, i, 0)),
                  _rows3(1),
                  pl.BlockSpec((2, 1, HID), lambda i: (0, 0, 0)),
                  _full2((HID, OUT_DIM)), _full2((HID, OUT_DIM))],
        out_specs=_rows3(OUT_DIM),
        out_shape=jax.ShapeDtypeStruct((2, N, OUT_DIM), jnp.float32),
    )(agg1a, agg1b, hs1, dinv, b1, W2a, W2b)

    agg2 = _sc_agg(srcL2, dstL2, hs2.reshape(2 * N, OUT_DIM))[:, :N]

    b2 = jnp.stack([b2a, b2b]).reshape(2, 1, OUT_DIM)
    logits, la, lb = pl.pallas_call(
        _tc3_body,
        grid=(_GRID,),
        in_specs=[_rows3(OUT_DIM), _rows3(OUT_DIM), _rows3(1),
                  pl.BlockSpec((2, 1, OUT_DIM), lambda i: (0, 0, 0))],
        out_specs=[_rows2(OUT_DIM)] * 3,
        out_shape=[jax.ShapeDtypeStruct((N, OUT_DIM), jnp.float32)] * 3,
    )(agg2, hs2, dinv, b2)

    return (logits, la, lb)


# speedup vs baseline: 2.2738x; 1.2345x over previous
import functools

import jax
import jax.numpy as jnp
from jax import lax
from jax.experimental import pallas as pl
from jax.experimental.pallas import tpu as pltpu
from jax.experimental.pallas import tpu_sc as plsc

N = 10000
E = 320000
IN_DIM = 128
HID = 128
OUT_DIM = 64
W64 = 64

C = 125
NCHUNK = E // C
NSUB = 16
CPT = NCHUNK // NSUB
CD = 80
CPTD = (E // CD) // NSUB
GC = 200
NSEG = 20000
ZPT = 632
NPAD = NSUB * ZPT

_MESH = plsc.VectorSubcoreMesh(core_axis_name="c", subcore_axis_name="s")
_SC_PARAMS = pltpu.CompilerParams(
    needs_layout_passes=False, use_tc_tiling_on_sc=False
)


@functools.partial(
    pl.kernel,
    out_type=jax.ShapeDtypeStruct((2, NSUB, NPAD), jnp.float32),
    mesh=_MESH,
    compiler_params=_SC_PARAMS,
    scratch_types=[
        pltpu.VMEM((CPTD, CD), jnp.int32),
        pltpu.VMEM((NPAD,), jnp.float32),
    ],
)
def _sc_degree(dst2_hbm, deg_hbm, idxd, hist):
    c = lax.axis_index("c")
    s = lax.axis_index("s")
    pltpu.sync_copy(dst2_hbm.at[c, s], idxd)

    def zero(r, _):
        hist[pl.ds(r * 16, 16)] = jnp.zeros((16,), jnp.float32)
        return ()
    lax.fori_loop(0, NPAD // 16, zero, ())

    ones16 = jnp.ones((16,), jnp.float32)

    def body(j, _):
        for k in range(CD // 16):
            idx = idxd[j, pl.ds(k * 16, 16)]
            plsc.addupdate_scatter(hist, [idx], ones16)
        return ()
    lax.fori_loop(0, CPTD, body, ())
    pltpu.sync_copy(hist, deg_hbm.at[c, s])


@functools.partial(
    pl.kernel,
    out_type=jax.ShapeDtypeStruct((2, NPAD, W64), jnp.float32),
    mesh=_MESH,
    compiler_params=_SC_PARAMS,
    scratch_types=[
        pltpu.VMEM((NSEG,), jnp.int32),
        pltpu.VMEM((NSEG,), jnp.int32),
        pltpu.VMEM((GC, W64), jnp.float32),
        pltpu.VMEM((GC, W64), jnp.float32),
        pltpu.VMEM_SHARED((NPAD, W64), jnp.float32),
        pltpu.SemaphoreType.DMA,
        pltpu.SemaphoreType.DMA,
    ],
)
def _sc_agg(src4_hbm, dst4_hbm, table_hbm, agg_hbm,
            idxs, idxd, rows0, rows1, acc, sem0, sem1):
    c = lax.axis_index("c")
    s = lax.axis_index("s")
    pltpu.sync_copy(src4_hbm.at[c, s], idxs)
    pltpu.sync_copy(dst4_hbm.at[c, s], idxd)

    def zrow(r, _):
        for k in range(W64 // 16):
            rows0[r, pl.ds(k * 16, 16)] = jnp.zeros((16,), jnp.float32)
        return ()
    lax.fori_loop(0, GC, zrow, ())
    nfull = ZPT // GC
    rem = ZPT - nfull * GC
    for k in range(nfull):
        pltpu.sync_copy(rows0, acc.at[pl.ds(s * ZPT + k * GC, GC)])
    pltpu.sync_copy(rows0.at[pl.ds(0, rem)],
                    acc.at[pl.ds(s * ZPT + nfull * GC, rem)])
    plsc.subcore_barrier()

    NT = NSEG // GC

    def start(j, buf, sem):
        pltpu.async_copy(table_hbm.at[idxs.at[pl.ds(j, GC)]], buf, sem)

    def drain(buf, sem):
        pltpu.make_async_copy(
            table_hbm.at[idxs.at[pl.ds(0, GC)]], buf, sem
        ).wait()

    start(0, rows0, sem0)
    start(GC, rows1, sem1)

    def body(p, _):
        t0 = 2 * p
        for buf, sem, t in ((rows0, sem0, t0), (rows1, sem1, t0 + 1)):
            drain(buf, sem)
            pltpu.sync_copy(buf, acc.at[idxd.at[pl.ds(t * GC, GC)]], add=True)
            start(jnp.minimum(t + 2, NT - 1) * GC, buf, sem)
        return ()
    lax.fori_loop(0, NT // 2, body, ())
    drain(rows0, sem0)
    drain(rows1, sem1)
    plsc.subcore_barrier()
    pltpu.sync_copy(acc.at[pl.ds(s * ZPT, ZPT)],
                    agg_hbm.at[c, pl.ds(s * ZPT, ZPT)])


def _tc1_body(deg_ref, x_ref, w1a_ref, w1b_ref, hs_ref, dinv_ref):
    deg = jnp.sum(deg_ref[...], axis=2)
    dinv = lax.rsqrt(deg + 1.0)[:, :, None]
    dinv_ref[...] = dinv
    x = x_ref[...]
    ha = dinv[0] * jnp.dot(x, w1a_ref[...], preferred_element_type=jnp.float32)
    hb = dinv[1] * jnp.dot(x, w1b_ref[...], preferred_element_type=jnp.float32)
    hs_ref[...] = jnp.stack([
        jnp.stack([ha[:, :W64], ha[:, W64:]]),
        jnp.stack([hb[:, :W64], hb[:, W64:]]),
    ])


def _tc2_body(agg1a_ref, agg1b_ref, hs1_ref, dinv_ref, b1_ref,
              w2a_ref, w2b_ref, hs2_ref):
    dinv = dinv_ref[...]
    agg_a = jnp.concatenate([agg1a_ref[0], agg1a_ref[1]], axis=-1)
    agg_b = jnp.concatenate([agg1b_ref[0], agg1b_ref[1]], axis=-1)
    hs_a = jnp.concatenate([hs1_ref[0, 0], hs1_ref[0, 1]], axis=-1)
    hs_b = jnp.concatenate([hs1_ref[1, 0], hs1_ref[1, 1]], axis=-1)
    xa = jax.nn.relu(dinv[0] * (agg_a + hs_a) + b1_ref[0])
    xb = jax.nn.relu(dinv[1] * (agg_b + hs_b) + b1_ref[1])
    ha = jnp.dot(xa, w2a_ref[...], preferred_element_type=jnp.float32)
    hb = jnp.dot(xb, w2b_ref[...], preferred_element_type=jnp.float32)
    hs2_ref[...] = jnp.stack([dinv[0] * ha, dinv[1] * hb])


def _tc3_body(agg_ref, hs_ref, dinv_ref, b2_ref, logits_ref, la_ref, lb_ref):
    l = dinv_ref[...] * (agg_ref[...] + hs_ref[...]) + b2_ref[...]
    la_ref[...] = l[0]
    lb_ref[...] = l[1]
    logits_ref[...] = 0.5 * l[0] + 0.5 * l[1]


_BLK = 1000
_GRID = N // _BLK


def _full2(shape):
    return pl.BlockSpec(shape, lambda i: (0, 0))


def _rows3(width):
    return pl.BlockSpec((2, _BLK, width), lambda i: (0, i, 0))


def _rows2(width):
    return pl.BlockSpec((_BLK, width), lambda i: (i, 0))


def _edges4(e0, e1):
    return jnp.stack([e0, e1]).reshape(2, NSUB, NSEG)


def _edges4d(e0, e1):
    return jnp.stack([e0, e1]).reshape(2, NSUB, CPTD, CD)


def kernel(x, edge_a, edge_b, W1a, b1a, W2a, b2a, W1b, b1b, W2b, b2b):
    sa, da = edge_a[0], edge_a[1]
    sb, db = edge_b[0], edge_b[1]
    srcL1a = _edges4(sa, sa + N)
    dstL1a = _edges4(da, da)
    srcL1b = _edges4(sb, sb + N)
    dstL1b = _edges4(db, db)
    srcL2 = _edges4(sa, sb + N)
    dstL2 = _edges4(da, db)
    dst2 = _edges4d(da, db)

    degp = _sc_degree(dst2)[:, :, :N].transpose(0, 2, 1)

    hs1, dinv = pl.pallas_call(
        _tc1_body,
        grid=(_GRID,),
        in_specs=[pl.BlockSpec((2, _BLK, NSUB), lambda i: (0, i, 0)),
                  _rows2(IN_DIM), _full2((IN_DIM, HID)),
                  _full2((IN_DIM, HID))],
        out_specs=[pl.BlockSpec((2, 2, _BLK, W64), lambda i: (0, 0, i, 0)),
                   _rows3(1)],
        out_shape=[jax.ShapeDtypeStruct((2, 2, N, W64), jnp.float32),
                   jax.ShapeDtypeStruct((2, N, 1), jnp.float32)],
    )(degp, x, W1a, W1b)

    agg1a = _sc_agg(srcL1a, dstL1a, hs1[0].reshape(2 * N, W64))[:, :N]
    agg1b = _sc_agg(srcL1b, dstL1b, hs1[1].reshape(2 * N, W64))[:, :N]

    b1 = jnp.stack([b1a, b1b]).reshape(2, 1, HID)
    hs2 = pl.pallas_call(
        _tc2_body,
        grid=(_GRID,),
        in_specs=[_rows3(W64), _rows3(W64),
                  pl.BlockSpec((2, 2, _BLK, W64), lambda i: (0, 0, i, 0)),
                  _rows3(1),
                  pl.BlockSpec((2, 1, HID), lambda i: (0, 0, 0)),
                  _full2((HID, OUT_DIM)), _full2((HID, OUT_DIM))],
        out_specs=_rows3(OUT_DIM),
        out_shape=jax.ShapeDtypeStruct((2, N, OUT_DIM), jnp.float32),
    )(agg1a, agg1b, hs1, dinv, b1, W2a, W2b)

    agg2 = _sc_agg(srcL2, dstL2, hs2.reshape(2 * N, OUT_DIM))[:, :N]

    b2 = jnp.stack([b2a, b2b]).reshape(2, 1, OUT_DIM)
    logits, la, lb = pl.pallas_call(
        _tc3_body,
        grid=(_GRID,),
        in_specs=[_rows3(OUT_DIM), _rows3(OUT_DIM), _rows3(1),
                  pl.BlockSpec((2, 1, OUT_DIM), lambda i: (0, 0, 0))],
        out_specs=[_rows2(OUT_DIM)] * 3,
        out_shape=[jax.ShapeDtypeStruct((N, OUT_DIM), jnp.float32)] * 3,
    )(agg2, hs2, dinv, b2)

    return (logits, la, lb)


# speedup vs baseline: 2.4566x; 1.0804x over previous
import functools

import jax
import jax.numpy as jnp
from jax import lax
from jax.experimental import pallas as pl
from jax.experimental.pallas import tpu as pltpu
from jax.experimental.pallas import tpu_sc as plsc

N = 10000
E = 320000
IN_DIM = 128
HID = 128
OUT_DIM = 64
W64 = 64

C = 125
NCHUNK = E // C
NSUB = 16
CPT = NCHUNK // NSUB
CD = 80
CPTD = (E // CD) // NSUB
GC = 200
NSEG = 20000
ZPT = 632
NPAD = NSUB * ZPT

_MESH = plsc.VectorSubcoreMesh(core_axis_name="c", subcore_axis_name="s")
_SC_PARAMS = pltpu.CompilerParams(
    needs_layout_passes=False, use_tc_tiling_on_sc=False
)


@functools.partial(
    pl.kernel,
    out_type=jax.ShapeDtypeStruct((2, NSUB, NPAD), jnp.float32),
    mesh=_MESH,
    compiler_params=_SC_PARAMS,
    scratch_types=[
        pltpu.VMEM((NSEG,), jnp.int32),
        pltpu.VMEM((NPAD,), jnp.float32),
    ],
)
def _sc_degree(e2_hbm, deg_hbm, idxd, hist):
    c = lax.axis_index("c")
    s = lax.axis_index("s")
    pltpu.sync_copy(e2_hbm.at[c, 1, pl.ds(s * NSEG, NSEG)], idxd)

    def zero(r, _):
        hist[pl.ds(r * 16, 16)] = jnp.zeros((16,), jnp.float32)
        return ()
    lax.fori_loop(0, NPAD // 16, zero, ())

    ones16 = jnp.ones((16,), jnp.float32)

    def body(j, _):
        for k in range(CD // 16):
            idx = idxd[pl.ds(j * CD + k * 16, 16)]
            plsc.addupdate_scatter(hist, [idx], ones16)
        return ()
    lax.fori_loop(0, NSEG // CD, body, ())
    pltpu.sync_copy(hist, deg_hbm.at[c, s])


def _make_sc_agg(edge_sel, table_sel, table_shape):
  @functools.partial(
      pl.kernel,
      out_type=jax.ShapeDtypeStruct((2, NPAD, W64), jnp.float32),
      mesh=_MESH,
      compiler_params=_SC_PARAMS,
      scratch_types=[
          pltpu.VMEM((NSEG,), jnp.int32),
          pltpu.VMEM((NSEG,), jnp.int32),
          pltpu.VMEM((GC, W64), jnp.float32),
          pltpu.VMEM((GC, W64), jnp.float32),
          pltpu.VMEM_SHARED((NPAD, W64), jnp.float32),
          pltpu.SemaphoreType.DMA,
          pltpu.SemaphoreType.DMA,
      ],
  )
  def _sc_agg(e2_hbm, full_table_hbm, agg_hbm,
              idxs, idxd, rows0, rows1, acc, sem0, sem1):
    c = lax.axis_index("c")
    s = lax.axis_index("s")
    src_sl, dst_sl = edge_sel(e2_hbm, c, s)
    table_hbm = table_sel(full_table_hbm, c)
    pltpu.sync_copy(src_sl, idxs)
    pltpu.sync_copy(dst_sl, idxd)

    def zrow(r, _):
        for k in range(W64 // 16):
            rows0[r, pl.ds(k * 16, 16)] = jnp.zeros((16,), jnp.float32)
        return ()
    lax.fori_loop(0, GC, zrow, ())
    nfull = ZPT // GC
    rem = ZPT - nfull * GC
    for k in range(nfull):
        pltpu.sync_copy(rows0, acc.at[pl.ds(s * ZPT + k * GC, GC)])
    pltpu.sync_copy(rows0.at[pl.ds(0, rem)],
                    acc.at[pl.ds(s * ZPT + nfull * GC, rem)])
    plsc.subcore_barrier()

    NT = NSEG // GC

    def start(j, buf, sem):
        pltpu.async_copy(table_hbm.at[idxs.at[pl.ds(j, GC)]], buf, sem)

    def drain(buf, sem):
        pltpu.make_async_copy(
            table_hbm.at[idxs.at[pl.ds(0, GC)]], buf, sem
        ).wait()

    start(0, rows0, sem0)
    start(GC, rows1, sem1)

    def body(p, _):
        t0 = 2 * p
        for buf, sem, t in ((rows0, sem0, t0), (rows1, sem1, t0 + 1)):
            drain(buf, sem)
            pltpu.sync_copy(buf, acc.at[idxd.at[pl.ds(t * GC, GC)]], add=True)
            start(jnp.minimum(t + 2, NT - 1) * GC, buf, sem)
        return ()
    lax.fori_loop(0, NT // 2, body, ())
    drain(rows0, sem0)
    drain(rows1, sem1)
    plsc.subcore_barrier()
    pltpu.sync_copy(acc.at[pl.ds(s * ZPT, ZPT)],
                    agg_hbm.at[c, pl.ds(s * ZPT, ZPT)])

  return _sc_agg


def _edges_of(b):
    def sel(e2, c, s):
        return (e2.at[b, 0, pl.ds(s * NSEG, NSEG)],
                e2.at[b, 1, pl.ds(s * NSEG, NSEG)])
    return sel


def _edges_by_core(e2, c, s):
    return (e2.at[c, 0, pl.ds(s * NSEG, NSEG)],
            e2.at[c, 1, pl.ds(s * NSEG, NSEG)])


_agg_l1a = _make_sc_agg(_edges_of(0), lambda t, c: t.at[0, c],
                        (2, 2, N, W64))
_agg_l1b = _make_sc_agg(_edges_of(1), lambda t, c: t.at[1, c],
                        (2, 2, N, W64))
_agg_l2 = _make_sc_agg(_edges_by_core, lambda t, c: t.at[c], (2, N, W64))


def _tc1_body(deg_ref, x_ref, w1a_ref, w1b_ref, hs_ref, dinv_ref):
    deg = jnp.sum(deg_ref[...], axis=2)
    dinv = lax.rsqrt(deg + 1.0)[:, :, None]
    dinv_ref[...] = dinv
    x = x_ref[...]
    ha = dinv[0] * jnp.dot(x, w1a_ref[...], preferred_element_type=jnp.float32)
    hb = dinv[1] * jnp.dot(x, w1b_ref[...], preferred_element_type=jnp.float32)
    hs_ref[...] = jnp.stack([
        jnp.stack([ha[:, :W64], ha[:, W64:]]),
        jnp.stack([hb[:, :W64], hb[:, W64:]]),
    ])


def _tc2_body(agg1a_ref, agg1b_ref, hs1_ref, dinv_ref, b1_ref,
              w2a_ref, w2b_ref, hs2_ref):
    dinv = dinv_ref[...]
    agg_a = jnp.concatenate([agg1a_ref[0], agg1a_ref[1]], axis=-1)
    agg_b = jnp.concatenate([agg1b_ref[0], agg1b_ref[1]], axis=-1)
    hs_a = jnp.concatenate([hs1_ref[0, 0], hs1_ref[0, 1]], axis=-1)
    hs_b = jnp.concatenate([hs1_ref[1, 0], hs1_ref[1, 1]], axis=-1)
    xa = jax.nn.relu(dinv[0] * (agg_a + hs_a) + b1_ref[0])
    xb = jax.nn.relu(dinv[1] * (agg_b + hs_b) + b1_ref[1])
    ha = jnp.dot(xa, w2a_ref[...], preferred_element_type=jnp.float32)
    hb = jnp.dot(xb, w2b_ref[...], preferred_element_type=jnp.float32)
    hs2_ref[...] = jnp.stack([dinv[0] * ha, dinv[1] * hb])


def _tc3_body(agg_ref, hs_ref, dinv_ref, b2_ref, logits_ref, la_ref, lb_ref):
    l = dinv_ref[...] * (agg_ref[...] + hs_ref[...]) + b2_ref[...]
    la_ref[...] = l[0]
    lb_ref[...] = l[1]
    logits_ref[...] = 0.5 * l[0] + 0.5 * l[1]


_BLK = 1000
_GRID = N // _BLK


def _full2(shape):
    return pl.BlockSpec(shape, lambda i: (0, 0))


def _rows3(width):
    return pl.BlockSpec((2, _BLK, width), lambda i: (0, i, 0))


def _rows2(width):
    return pl.BlockSpec((_BLK, width), lambda i: (i, 0))


def kernel(x, edge_a, edge_b, W1a, b1a, W2a, b2a, W1b, b1b, W2b, b2b):
    e2 = jnp.stack([edge_a, edge_b])

    degp = _sc_degree(e2)[:, :, :N].transpose(0, 2, 1)

    hs1, dinv = pl.pallas_call(
        _tc1_body,
        grid=(_GRID,),
        in_specs=[pl.BlockSpec((2, _BLK, NSUB), lambda i: (0, i, 0)),
                  _rows2(IN_DIM), _full2((IN_DIM, HID)),
                  _full2((IN_DIM, HID))],
        out_specs=[pl.BlockSpec((2, 2, _BLK, W64), lambda i: (0, 0, i, 0)),
                   _rows3(1)],
        out_shape=[jax.ShapeDtypeStruct((2, 2, N, W64), jnp.float32),
                   jax.ShapeDtypeStruct((2, N, 1), jnp.float32)],
    )(degp, x, W1a, W1b)

    agg1a = _agg_l1a(e2, hs1)[:, :N]
    agg1b = _agg_l1b(e2, hs1)[:, :N]

    b1 = jnp.stack([b1a, b1b]).reshape(2, 1, HID)
    hs2 = pl.pallas_call(
        _tc2_body,
        grid=(_GRID,),
        in_specs=[_rows3(W64), _rows3(W64),
                  pl.BlockSpec((2, 2, _BLK, W64), lambda i: (0, 0, i, 0)),
                  _rows3(1),
                  pl.BlockSpec((2, 1, HID), lambda i: (0, 0, 0)),
                  _full2((HID, OUT_DIM)), _full2((HID, OUT_DIM))],
        out_specs=_rows3(OUT_DIM),
        out_shape=jax.ShapeDtypeStruct((2, N, OUT_DIM), jnp.float32),
    )(agg1a, agg1b, hs1, dinv, b1, W2a, W2b)

    agg2 = _agg_l2(e2, hs2)[:, :N]

    b2 = jnp.stack([b2a, b2b]).reshape(2, 1, OUT_DIM)
    logits, la, lb = pl.pallas_call(
        _tc3_body,
        grid=(_GRID,),
        in_specs=[_rows3(OUT_DIM), _rows3(OUT_DIM), _rows3(1),
                  pl.BlockSpec((2, 1, OUT_DIM), lambda i: (0, 0, 0))],
        out_specs=[_rows2(OUT_DIM)] * 3,
        out_shape=[jax.ShapeDtypeStruct((N, OUT_DIM), jnp.float32)] * 3,
    )(agg2, hs2, dinv, b2)

    return (logits, la, lb)
